# baseline (speedup 1.0000x reference)
import jax
import jax.numpy as jnp
from jax.experimental import pallas as pl
from jax.experimental.pallas import tpu as pltpu

_D_MODEL = 2048
_N_EXPERTS = 64
_TB = 2048
_KS = 2
_KC = _D_MODEL // _KS


def _router_body(x_ref, w_ref, w_out_ref, e_out_ref, acc_ref):
    k = pl.program_id(1)
    part = jax.lax.dot_general(
        x_ref[...],
        w_ref[:, pl.ds(k * _KC, _KC)],
        dimension_numbers=(((1,), (1,)), ((), ())),
        preferred_element_type=jnp.float32,
    )

    @pl.when(k == 0)
    def _():
        acc_ref[...] = part

    @pl.when(k == _KS - 1)
    def _():
        logits = acc_ref[...] + part
        iota = jax.lax.broadcasted_iota(jnp.int32, logits.shape, 1)

        m1 = jnp.max(logits, axis=1, keepdims=True)
        idx1 = jnp.argmax(logits, axis=1, keepdims=True)
        masked = jnp.where(iota == idx1, -jnp.inf, logits)
        m2 = jnp.max(masked, axis=1, keepdims=True)
        idx2 = jnp.argmax(masked, axis=1, keepdims=True)

        w1 = jax.nn.sigmoid(m1 - m2)
        w_out_ref[...] = jnp.concatenate([w1, 1.0 - w1], axis=1)
        e_out_ref[...] = jnp.concatenate([idx1, idx2], axis=1)


def kernel(x, W):
    n_tokens = x.shape[0]
    grid = (n_tokens // _TB, _KS)
    weights, experts = pl.pallas_call(
        _router_body,
        grid=grid,
        in_specs=[
            pl.BlockSpec((_TB, _KC), lambda i, k: (i, k)),
            pl.BlockSpec((_N_EXPERTS, _D_MODEL), lambda i, k: (0, 0)),
        ],
        out_specs=[
            pl.BlockSpec((_TB, 2), lambda i, k: (i, 0)),
            pl.BlockSpec((_TB, 2), lambda i, k: (i, 0)),
        ],
        out_shape=[
            jax.ShapeDtypeStruct((n_tokens, 2), jnp.float32),
            jax.ShapeDtypeStruct((n_tokens, 2), jnp.int32),
        ],
        scratch_shapes=[pltpu.VMEM((_TB, _N_EXPERTS), jnp.float32)],
        compiler_params=pltpu.CompilerParams(
            dimension_semantics=("parallel", "arbitrary"),
        ),
    )(x, W)
    return (weights, experts)


# speedup vs baseline: 1.1107x; 1.1107x over previous
import jax
import jax.numpy as jnp
from jax.experimental import pallas as pl
from jax.experimental.pallas import tpu as pltpu

_D_MODEL = 2048
_N_EXPERTS = 64
_N_TOKENS = 16384
_TB = 2048


def _router_body(x_ref, w_ref, w_out_ref, e_out_ref):
    logits = jax.lax.dot_general(
        x_ref[...],
        w_ref[...],
        dimension_numbers=(((1,), (1,)), ((), ())),
        preferred_element_type=jnp.float32,
    )
    iota = jax.lax.broadcasted_iota(jnp.int32, logits.shape, 1)

    m1 = jnp.max(logits, axis=1, keepdims=True)
    idx1 = jnp.argmax(logits, axis=1, keepdims=True)
    masked = jnp.where(iota == idx1, -jnp.inf, logits)
    m2 = jnp.max(masked, axis=1, keepdims=True)
    idx2 = jnp.argmax(masked, axis=1, keepdims=True)

    w1 = jax.nn.sigmoid(m1 - m2)
    w_out_ref[...] = jnp.concatenate([w1, 1.0 - w1], axis=1)
    e_out_ref[...] = jnp.concatenate([idx1, idx2], axis=1)


def kernel(x, W):
    n_tokens = x.shape[0]
    grid = (n_tokens // _TB,)
    weights, experts = pl.pallas_call(
        _router_body,
        grid=grid,
        in_specs=[
            pl.BlockSpec((_TB, _D_MODEL), lambda i: (i, 0)),
            pl.BlockSpec((_N_EXPERTS, _D_MODEL), lambda i: (0, 0)),
        ],
        out_specs=[
            pl.BlockSpec((_TB, 2), lambda i: (i, 0)),
            pl.BlockSpec((_TB, 2), lambda i: (i, 0)),
        ],
        out_shape=[
            jax.ShapeDtypeStruct((n_tokens, 2), jnp.float32),
            jax.ShapeDtypeStruct((n_tokens, 2), jnp.int32),
        ],
        compiler_params=pltpu.CompilerParams(
            dimension_semantics=("parallel",),
        ),
    )(x, W)
    return (weights, experts)


# speedup vs baseline: 1.1110x; 1.0002x over previous
import jax
import jax.numpy as jnp
from jax.experimental import pallas as pl
from jax.experimental.pallas import tpu as pltpu

_D_MODEL = 2048
_N_EXPERTS = 64
_TB = 2048


def _router_body(x_ref, w_ref, w_out_ref, e_out_ref):
    logits = jax.lax.dot_general(
        x_ref[...],
        w_ref[...],
        dimension_numbers=(((1,), (1,)), ((), ())),
        preferred_element_type=jnp.float32,
    )
    iota = jax.lax.broadcasted_iota(jnp.int32, logits.shape, 1)

    m1 = jnp.max(logits, axis=1, keepdims=True)
    idx1 = jnp.argmax(logits, axis=1, keepdims=True)
    masked = jnp.where(iota == idx1, -jnp.inf, logits)
    m2 = jnp.max(masked, axis=1, keepdims=True)
    idx2 = jnp.argmax(masked, axis=1, keepdims=True)

    w1 = jax.nn.sigmoid(m1 - m2)
    w_out_ref[...] = jnp.concatenate([w1, 1.0 - w1], axis=1)
    e_out_ref[...] = jnp.concatenate([idx1, idx2], axis=1)


def kernel(x, W):
    n_tokens = x.shape[0]
    grid = (n_tokens // _TB,)
    weights, experts = pl.pallas_call(
        _router_body,
        grid=grid,
        in_specs=[
            pl.BlockSpec((_TB, _D_MODEL), lambda i: (i, 0)),
            pl.BlockSpec((_N_EXPERTS, _D_MODEL), lambda i: (0, 0)),
        ],
        out_specs=[
            pl.BlockSpec((_TB, 2), lambda i: (i, 0)),
            pl.BlockSpec((_TB, 2), lambda i: (i, 0)),
        ],
        out_shape=[
            jax.ShapeDtypeStruct((n_tokens, 2), jnp.float32),
            jax.ShapeDtypeStruct((n_tokens, 2), jnp.int32),
        ],
        compiler_params=pltpu.CompilerParams(
            dimension_semantics=("parallel",),
        ),
    )(x, W)
    return (weights, experts)
